# baseline (speedup 1.0000x reference)
import jax
import jax.numpy as jnp
from jax.experimental import pallas as pl

_BT = 512


def _router_body(x_ref, w_ref, b_ref, o_ref):
    logits = jax.lax.dot_general(
        w_ref[...], x_ref[...],
        dimension_numbers=(((1,), (1,)), ((), ())),
        preferred_element_type=jnp.float32,
    ) + b_ref[...][:, None]
    m = jnp.max(logits, axis=0, keepdims=True)
    e = jnp.exp(logits - m)
    o_ref[...] = e / jnp.sum(e, axis=0, keepdims=True)


@jax.jit
def kernel(x, W, b):
    n_tokens, embed_dim = x.shape
    n_experts = W.shape[0]
    grid = (n_tokens // _BT,)
    out_t = pl.pallas_call(
        _router_body,
        grid=grid,
        in_specs=[
            pl.BlockSpec((_BT, embed_dim), lambda i: (i, 0)),
            pl.BlockSpec((n_experts, embed_dim), lambda i: (0, 0)),
            pl.BlockSpec((n_experts,), lambda i: (0,)),
        ],
        out_specs=pl.BlockSpec((n_experts, _BT), lambda i: (0, i)),
        out_shape=jax.ShapeDtypeStruct((n_experts, n_tokens), jnp.float32),
    )(x, W, b)
    return out_t.T


# speedup vs baseline: 1.1951x; 1.1951x over previous
import jax
import jax.numpy as jnp
from jax.experimental import pallas as pl

_BT = 1024


def _router_body(x_ref, w_ref, b_ref, o_ref):
    logits = jax.lax.dot_general(
        w_ref[...], x_ref[...],
        dimension_numbers=(((1,), (1,)), ((), ())),
        preferred_element_type=jnp.float32,
    ) + b_ref[...][:, None]
    m = jnp.max(logits, axis=0, keepdims=True)
    e = jnp.exp(logits - m)
    o_ref[...] = e / jnp.sum(e, axis=0, keepdims=True)


@jax.jit
def kernel(x, W, b):
    n_tokens, embed_dim = x.shape
    n_experts = W.shape[0]
    grid = (n_tokens // _BT,)
    out_t = pl.pallas_call(
        _router_body,
        grid=grid,
        in_specs=[
            pl.BlockSpec((_BT, embed_dim), lambda i: (i, 0)),
            pl.BlockSpec((n_experts, embed_dim), lambda i: (0, 0)),
            pl.BlockSpec((n_experts,), lambda i: (0,)),
        ],
        out_specs=pl.BlockSpec((n_experts, _BT), lambda i: (0, i)),
        out_shape=jax.ShapeDtypeStruct((n_experts, n_tokens), jnp.float32),
    )(x, W, b)
    return out_t.T
